# Initial kernel scaffold; baseline (speedup 1.0000x reference)
import jax
import jax.numpy as jnp
from jax.experimental import pallas as pl


def kernel(x_user, x_item, ei_rates, ei_rated_by, ei_similar, Wu1, bu1, Wu2, bu2, Wi1, bi1, Wi2, bi2, Wr_rates, Wr_ratedby, Wr_similar, a_item, Wc1, bc1, Wc2, bc2):
    raise NotImplementedError("write your pallas kernel here")


# speedup vs baseline: 3.5322x; 3.5322x over previous
import functools

import jax
import jax.numpy as jnp
from jax import lax
from jax.experimental import pallas as pl
from jax.experimental.pallas import tpu as pltpu
from jax.experimental.pallas import tpu_sc as plsc

M = 50000
E = 400000
D = 128
H = 128
OUT = 64
NCHUNK = 8
CW = 16
RPN = H // CW
NT = 16
BATCH = 128
KINF = 5
EPTN = 24992
EPT0 = EPTN + BATCH
NGROUP = 39
REM = 32
CROWS = 3128
CROWS_L = M - (NT - 1) * CROWS
ZROWS = 400
ZSPLIT = (400,) * 7 + (328,)
ZSPLIT_L = (400,) * 7 + (280,)


def _gelu(x):
    return 0.5 * x * (1.0 + lax.erf(x * 0.7071067811865476))


def _enc_body(x_ref, w1_ref, b1_ref, w2_ref, b2_ref, o_ref):
    h1 = _gelu(
        jnp.dot(x_ref[0], w1_ref[0], preferred_element_type=jnp.float32)
        + b1_ref[0]
    )
    o_ref[0] = (
        jnp.dot(h1, w2_ref[0], preferred_element_type=jnp.float32) + b2_ref[0]
    )


def _encode(xs, w1s, b1s, w2s, b2s, bm):
    nb = M // bm
    return pl.pallas_call(
        _enc_body,
        grid=(2, nb),
        in_specs=[
            pl.BlockSpec((1, bm, D), lambda t, i: (t, i, 0)),
            pl.BlockSpec((1, D, H), lambda t, i: (t, 0, 0)),
            pl.BlockSpec((1, 1, H), lambda t, i: (t, 0, 0)),
            pl.BlockSpec((1, H, H), lambda t, i: (t, 0, 0)),
            pl.BlockSpec((1, 1, H), lambda t, i: (t, 0, 0)),
        ],
        out_specs=pl.BlockSpec((1, bm, H), lambda t, i: (t, i, 0)),
        out_shape=jax.ShapeDtypeStruct((2, M, H), jnp.float32),
    )(xs, w1s, b1s, w2s, b2s)


def _agg_body(h2_ref, eir_ref, eis_ref, z2d_ref, z1d_ref, o1d_ref,
              sum1_ref, sum2_ref, cnt1_ref, cnt2_ref,
              src_buf, dst_buf, rows,
              gidx0, gidx1, gidx2, gidx3, gidx4,
              didx0, didx1, didx2, didx3, didx4,
              gidx_r, didx_r, rows_r, ones_buf, zero_buf, zc_buf,
              acc, cnt_acc, sem):
    gidx = [gidx0, gidx1, gidx2, gidx3, gidx4]
    didx = [didx0, didx1, didx2, didx3, didx4]
    core = lax.axis_index("c")
    s = lax.axis_index("s")
    row0 = s * CROWS
    e0 = jnp.where(s == 0, 0, BATCH + s * EPTN)

    pltpu.sync_copy(z2d_ref, zero_buf)
    pltpu.sync_copy(z1d_ref, zc_buf)
    pltpu.sync_copy(o1d_ref, ones_buf)

    def build_idx(gi_ref, di_ref, base, base_g):
        for q in range(BATCH // 16):
            sv = src_buf[pl.ds(base + 16 * q, 16)]
            gi_ref[pl.ds(16 * q, 16)] = sv * RPN + base_g
            di_ref[pl.ds(16 * q, 16)] = dst_buf[pl.ds(base + 16 * q, 16)]

    for p in range(2 * (NCHUNK // 2)):
        rel = p // (NCHUNK // 2)
        ei_ref = eir_ref if rel == 0 else eis_ref
        sum_ref = sum1_ref if rel == 0 else sum2_ref
        chunk = (NCHUNK // 2) * core + (p % (NCHUNK // 2))
        base_g = rel * (RPN * M) + chunk
        if p == 0:
            cnt_cond, cnt_out = (core == 0), cnt1_ref
        elif p == NCHUNK // 2:
            cnt_cond, cnt_out = (core == 1), cnt2_ref
        else:
            cnt_cond, cnt_out = None, None

        @pl.when(s < NT - 1)
        def _():
            off = 0
            for zsz in ZSPLIT:
                pltpu.sync_copy(zero_buf.at[pl.ds(0, zsz)],
                                acc.at[pl.ds(row0 + off, zsz)])
                off += zsz

        @pl.when(s == NT - 1)
        def _():
            off = 0
            for zsz in ZSPLIT_L:
                pltpu.sync_copy(zero_buf.at[pl.ds(0, zsz)],
                                acc.at[pl.ds(row0 + off, zsz)])
                off += zsz

        if cnt_cond is not None:
            @pl.when(cnt_cond & (s < NT - 1))
            def _():
                pltpu.sync_copy(zc_buf, cnt_acc.at[pl.ds(row0, CROWS)])

            @pl.when(cnt_cond & (s == NT - 1))
            def _():
                pltpu.sync_copy(zc_buf.at[pl.ds(0, CROWS_L)],
                                cnt_acc.at[pl.ds(row0, CROWS_L)])
        plsc.subcore_barrier()

        if p % (NCHUNK // 2) == 0:
            @pl.when(s == 0)
            def _():
                pltpu.sync_copy(ei_ref.at[pl.ds(0, EPT0)], src_buf)
                pltpu.sync_copy(ei_ref.at[pl.ds(E, EPT0)], dst_buf)

            @pl.when(s > 0)
            def _():
                pltpu.sync_copy(ei_ref.at[pl.ds(e0, EPTN)],
                                src_buf.at[pl.ds(0, EPTN)])
                pltpu.sync_copy(ei_ref.at[pl.ds(E + e0, EPTN)],
                                dst_buf.at[pl.ds(0, EPTN)])

        def group(g, carry):
            base = g * (KINF * BATCH)
            handles = []
            for k in range(KINF):
                build_idx(gidx[k], didx[k], base + k * BATCH, base_g)
                handles.append(
                    pltpu.async_copy(h2_ref.at[gidx[k]], rows.at[k], sem))
            for k in range(KINF):
                handles[k].wait()
                pltpu.sync_copy(rows.at[k], acc.at[didx[k]], add=True)
                if cnt_cond is not None:
                    @pl.when(cnt_cond)
                    def _():
                        pltpu.sync_copy(ones_buf, cnt_acc.at[didx[k]],
                                        add=True)
            return carry

        lax.fori_loop(0, NGROUP, group, 0)

        @pl.when(s == 0)
        def _():
            build_idx(gidx[0], didx[0], NGROUP * KINF * BATCH, base_g)
            pltpu.async_copy(h2_ref.at[gidx[0]], rows.at[0], sem).wait()
            pltpu.sync_copy(rows.at[0], acc.at[didx[0]], add=True)
            if cnt_cond is not None:
                @pl.when(cnt_cond)
                def _():
                    pltpu.sync_copy(ones_buf, cnt_acc.at[didx[0]], add=True)

        rbase = jnp.where(s == 0, EPT0 - REM, EPTN - REM)
        for q in range(REM // 16):
            sv = src_buf[pl.ds(rbase + 16 * q, 16)]
            gidx_r[pl.ds(16 * q, 16)] = sv * RPN + base_g
            didx_r[pl.ds(16 * q, 16)] = dst_buf[pl.ds(rbase + 16 * q, 16)]
        pltpu.async_copy(h2_ref.at[gidx_r], rows_r, sem).wait()
        pltpu.sync_copy(rows_r, acc.at[didx_r], add=True)
        if cnt_cond is not None:
            @pl.when(cnt_cond)
            def _():
                pltpu.sync_copy(ones_buf.at[pl.ds(0, REM)],
                                cnt_acc.at[didx_r], add=True)
        plsc.subcore_barrier()

        for half in (0, 1):
            ch = (NCHUNK // 2) * half + (p % (NCHUNK // 2))

            @pl.when((core == half) & (s < NT - 1))
            def _(ch=ch):
                pltpu.sync_copy(acc.at[pl.ds(row0, CROWS)],
                                sum_ref.at[ch, pl.ds(row0, CROWS)])

            @pl.when((core == half) & (s == NT - 1))
            def _(ch=ch):
                pltpu.sync_copy(acc.at[pl.ds(row0, CROWS_L)],
                                sum_ref.at[ch, pl.ds(row0, CROWS_L)])

        if cnt_cond is not None:
            @pl.when(cnt_cond & (s < NT - 1))
            def _():
                pltpu.sync_copy(cnt_acc.at[pl.ds(row0, CROWS)],
                                cnt_out.at[pl.ds(row0, CROWS)])

            @pl.when(cnt_cond & (s == NT - 1))
            def _():
                pltpu.sync_copy(cnt_acc.at[pl.ds(row0, CROWS_L)],
                                cnt_out.at[pl.ds(row0, CROWS_L)])
        plsc.subcore_barrier()


def _aggregate(h2, ei_rates, ei_similar):
    z2d = jnp.zeros((ZROWS, CW), jnp.float32)
    z1d = jnp.zeros((CROWS,), jnp.float32)
    o1d = jnp.ones((BATCH,), jnp.float32)
    mesh = plsc.VectorSubcoreMesh(core_axis_name="c", subcore_axis_name="s")
    scratch = [
        pltpu.VMEM((EPT0,), jnp.int32),
        pltpu.VMEM((EPT0,), jnp.int32),
        pltpu.VMEM((KINF, BATCH, CW), jnp.float32),
    ] + [pltpu.VMEM((BATCH,), jnp.int32) for _ in range(2 * KINF)] + [
        pltpu.VMEM((REM,), jnp.int32),
        pltpu.VMEM((REM,), jnp.int32),
        pltpu.VMEM((REM, CW), jnp.float32),
        pltpu.VMEM((BATCH,), jnp.float32),
        pltpu.VMEM((ZROWS, CW), jnp.float32),
        pltpu.VMEM((CROWS,), jnp.float32),
        pltpu.VMEM_SHARED((M, CW), jnp.float32),
        pltpu.VMEM_SHARED((M,), jnp.float32),
        pltpu.SemaphoreType.DMA,
    ]
    out_type = [
        jax.ShapeDtypeStruct((NCHUNK, M, CW), jnp.float32),
        jax.ShapeDtypeStruct((NCHUNK, M, CW), jnp.float32),
        jax.ShapeDtypeStruct((M,), jnp.float32),
        jax.ShapeDtypeStruct((M,), jnp.float32),
    ]
    f = pl.kernel(_agg_body, mesh=mesh, out_type=out_type,
                  scratch_types=scratch,
                  compiler_params=pltpu.CompilerParams(
                      use_tc_tiling_on_sc=False))
    return f(h2, ei_rates.reshape(2 * E), ei_similar.reshape(2 * E),
             z2d, z1d, o1d)


def _comb_body(s1_ref, c1_ref, s2_ref, c2_ref, h_ref, wr1_ref, wr2_ref,
               a_ref, wc1_ref, bc1_ref, wc2_ref, bc2_ref, o_ref):
    acc1 = jnp.dot(s1_ref[0], wr1_ref[0], preferred_element_type=jnp.float32)
    acc2 = jnp.dot(s2_ref[0], wr2_ref[0], preferred_element_type=jnp.float32)
    for c in range(1, NCHUNK):
        acc1 += jnp.dot(s1_ref[c], wr1_ref[c],
                        preferred_element_type=jnp.float32)
        acc2 += jnp.dot(s2_ref[c], wr2_ref[c],
                        preferred_element_type=jnp.float32)
    cnt1 = c1_ref[...]
    cnt2 = c2_ref[...]
    msg1 = jnp.where(cnt1 > 0, acc1 / jnp.maximum(cnt1, 1.0), 0.0)
    msg2 = jnp.where(cnt2 > 0, acc2 / jnp.maximum(cnt2, 1.0), 0.0)
    a1 = jnp.dot(msg1, a_ref[...], preferred_element_type=jnp.float32)
    a2 = jnp.dot(msg2, a_ref[...], preferred_element_type=jnp.float32)
    mx = jnp.maximum(a1, a2)
    e1 = jnp.exp(a1 - mx)
    e2 = jnp.exp(a2 - mx)
    inv = 1.0 / (e1 + e2)
    hcomb = msg1 * (e1 * inv) + msg2 * (e2 * inv) + h_ref[0]
    t1 = _gelu(
        jnp.dot(hcomb, wc1_ref[...], preferred_element_type=jnp.float32)
        + bc1_ref[...]
    )
    o_ref[...] = (
        jnp.dot(t1, wc2_ref[...], preferred_element_type=jnp.float32)
        + bc2_ref[...]
    )


def _combine(sum1, cnt1, sum2, cnt2, h, wr1, wr2, a_item, wc1, bc1, wc2, bc2,
             bm):
    nb = M // bm
    return pl.pallas_call(
        _comb_body,
        grid=(nb,),
        in_specs=[
            pl.BlockSpec((NCHUNK, bm, CW), lambda i: (0, i, 0)),
            pl.BlockSpec((bm, 1), lambda i: (i, 0)),
            pl.BlockSpec((NCHUNK, bm, CW), lambda i: (0, i, 0)),
            pl.BlockSpec((bm, 1), lambda i: (i, 0)),
            pl.BlockSpec((1, bm, H), lambda i: (1, i, 0)),
            pl.BlockSpec((NCHUNK, CW, H), lambda i: (0, 0, 0)),
            pl.BlockSpec((NCHUNK, CW, H), lambda i: (0, 0, 0)),
            pl.BlockSpec((H, 1), lambda i: (0, 0)),
            pl.BlockSpec((H, H), lambda i: (0, 0)),
            pl.BlockSpec((1, H), lambda i: (0, 0)),
            pl.BlockSpec((H, OUT), lambda i: (0, 0)),
            pl.BlockSpec((1, OUT), lambda i: (0, 0)),
        ],
        out_specs=pl.BlockSpec((bm, OUT), lambda i: (i, 0)),
        out_shape=jax.ShapeDtypeStruct((M, OUT), jnp.float32),
    )(sum1, cnt1, sum2, cnt2, h, wr1, wr2, a_item, wc1, bc1, wc2, bc2)


def kernel(x_user, x_item, ei_rates, ei_rated_by, ei_similar,
           Wu1, bu1, Wu2, bu2, Wi1, bi1, Wi2, bi2,
           Wr_rates, Wr_ratedby, Wr_similar, a_item,
           Wc1, bc1, Wc2, bc2):
    del ei_rated_by, Wr_ratedby
    xs = jnp.stack([x_user, x_item])
    w1s = jnp.stack([Wu1, Wi1])
    b1s = jnp.stack([bu1, bi1])[:, None, :]
    w2s = jnp.stack([Wu2, Wi2])
    b2s = jnp.stack([bu2, bi2])[:, None, :]
    h = _encode(xs, w1s, b1s, w2s, b2s, bm=2000)
    h2 = h.reshape(2 * M * (H // CW), CW)
    sum1, sum2, cnt1, cnt2 = _aggregate(h2, ei_rates, ei_similar)
    logits = _combine(
        sum1, cnt1.reshape(M, 1), sum2, cnt2.reshape(M, 1), h,
        Wr_rates.reshape(NCHUNK, CW, H), Wr_similar.reshape(NCHUNK, CW, H),
        a_item, Wc1, bc1.reshape(1, H), Wc2, bc2.reshape(1, OUT), bm=2000)
    return logits


# speedup vs baseline: 4.6822x; 1.3256x over previous
import jax
import jax.numpy as jnp
from jax import lax
from jax.experimental import pallas as pl
from jax.experimental.pallas import tpu as pltpu
from jax.experimental.pallas import tpu_sc as plsc

M = 50000
M2 = 51200
E = 400000
D = 128
H = 128
OUT = 64
NCHUNK = 8
CW = 16
RPN = H // CW
NT = 16
BATCH = 128
KINF = 5
EPTN = 24992
EPT0 = EPTN + BATCH
NGROUP = 39
REM = 32
CROWS = 3128
CROWS_L = M - (NT - 1) * CROWS
ZROWS = 400
ZSPLIT = (400,) * 7 + (328,)
ZSPLIT_L = (400,) * 7 + (280,)
BM = 2048


def _gelu(x):
    return 0.5 * x * (1.0 + lax.erf(x * 0.7071067811865476))


def _enc_body(x_ref, w1_ref, b1_ref, w2_ref, b2_ref, o_ref):
    h1 = _gelu(
        jnp.dot(x_ref[...], w1_ref[...], preferred_element_type=jnp.float32)
        + b1_ref[...]
    )
    o_ref[...] = (
        jnp.dot(h1, w2_ref[...], preferred_element_type=jnp.float32)
        + b2_ref[...]
    )


def _encode(x, w1, b1, w2, b2):
    return pl.pallas_call(
        _enc_body,
        grid=(M2 // BM,),
        in_specs=[
            pl.BlockSpec((BM, D), lambda i: (i, 0)),
            pl.BlockSpec((D, H), lambda i: (0, 0)),
            pl.BlockSpec((1, H), lambda i: (0, 0)),
            pl.BlockSpec((H, H), lambda i: (0, 0)),
            pl.BlockSpec((1, H), lambda i: (0, 0)),
        ],
        out_specs=pl.BlockSpec((BM, H), lambda i: (i, 0)),
        out_shape=jax.ShapeDtypeStruct((M2, H), jnp.float32),
    )(x, w1, b1.reshape(1, H), w2, b2.reshape(1, H))


def _agg_body(hu_ref, hi_ref, eir_ref, eis_ref, z2d_ref, z1d_ref, o1d_ref,
              sum1_ref, sum2_ref, cnt1_ref, cnt2_ref,
              src_buf, dst_buf, rows,
              gidx0, gidx1, gidx2, gidx3, gidx4,
              didx0, didx1, didx2, didx3, didx4,
              gidx_r, didx_r, rows_r, ones_buf, zero_buf, zc_buf,
              acc, cnt_acc, sem):
    gidx = [gidx0, gidx1, gidx2, gidx3, gidx4]
    didx = [didx0, didx1, didx2, didx3, didx4]
    core = lax.axis_index("c")
    s = lax.axis_index("s")
    row0 = s * CROWS
    e0 = jnp.where(s == 0, 0, BATCH + s * EPTN)

    pltpu.sync_copy(z2d_ref, zero_buf)
    pltpu.sync_copy(z1d_ref, zc_buf)
    pltpu.sync_copy(o1d_ref, ones_buf)

    def build_idx(gi_ref, di_ref, base, base_g):
        for q in range(BATCH // 16):
            sv = src_buf[pl.ds(base + 16 * q, 16)]
            gi_ref[pl.ds(16 * q, 16)] = sv * RPN + base_g
            di_ref[pl.ds(16 * q, 16)] = dst_buf[pl.ds(base + 16 * q, 16)]

    for p in range(2 * (NCHUNK // 2)):
        rel = p // (NCHUNK // 2)
        ei_ref = eir_ref if rel == 0 else eis_ref
        h_ref = hu_ref if rel == 0 else hi_ref
        sum_ref = sum1_ref if rel == 0 else sum2_ref
        chunk = (NCHUNK // 2) * core + (p % (NCHUNK // 2))
        base_g = chunk
        if p == 0:
            cnt_cond, cnt_out = (core == 0), cnt1_ref
        elif p == NCHUNK // 2:
            cnt_cond, cnt_out = (core == 1), cnt2_ref
        else:
            cnt_cond, cnt_out = None, None

        @pl.when(s < NT - 1)
        def _():
            off = 0
            for zsz in ZSPLIT:
                pltpu.sync_copy(zero_buf.at[pl.ds(0, zsz)],
                                acc.at[pl.ds(row0 + off, zsz)])
                off += zsz

        @pl.when(s == NT - 1)
        def _():
            off = 0
            for zsz in ZSPLIT_L:
                pltpu.sync_copy(zero_buf.at[pl.ds(0, zsz)],
                                acc.at[pl.ds(row0 + off, zsz)])
                off += zsz

        if cnt_cond is not None:
            @pl.when(cnt_cond & (s < NT - 1))
            def _():
                pltpu.sync_copy(zc_buf, cnt_acc.at[pl.ds(row0, CROWS)])

            @pl.when(cnt_cond & (s == NT - 1))
            def _():
                pltpu.sync_copy(zc_buf.at[pl.ds(0, CROWS_L)],
                                cnt_acc.at[pl.ds(row0, CROWS_L)])
        plsc.subcore_barrier()

        if p % (NCHUNK // 2) == 0:
            @pl.when(s == 0)
            def _():
                pltpu.sync_copy(ei_ref.at[pl.ds(0, EPT0)], src_buf)
                pltpu.sync_copy(ei_ref.at[pl.ds(E, EPT0)], dst_buf)

            @pl.when(s > 0)
            def _():
                pltpu.sync_copy(ei_ref.at[pl.ds(e0, EPTN)],
                                src_buf.at[pl.ds(0, EPTN)])
                pltpu.sync_copy(ei_ref.at[pl.ds(E + e0, EPTN)],
                                dst_buf.at[pl.ds(0, EPTN)])

        def group(g, carry):
            base = g * (KINF * BATCH)
            handles = []
            for k in range(KINF):
                build_idx(gidx[k], didx[k], base + k * BATCH, base_g)
                handles.append(
                    pltpu.async_copy(h_ref.at[gidx[k]], rows.at[k], sem))
            for k in range(KINF):
                handles[k].wait()
                pltpu.sync_copy(rows.at[k], acc.at[didx[k]], add=True)
                if cnt_cond is not None:
                    @pl.when(cnt_cond)
                    def _():
                        pltpu.sync_copy(ones_buf, cnt_acc.at[didx[k]],
                                        add=True)
            return carry

        lax.fori_loop(0, NGROUP, group, 0)

        @pl.when(s == 0)
        def _():
            build_idx(gidx[0], didx[0], NGROUP * KINF * BATCH, base_g)
            pltpu.async_copy(h_ref.at[gidx[0]], rows.at[0], sem).wait()
            pltpu.sync_copy(rows.at[0], acc.at[didx[0]], add=True)
            if cnt_cond is not None:
                @pl.when(cnt_cond)
                def _():
                    pltpu.sync_copy(ones_buf, cnt_acc.at[didx[0]], add=True)

        rbase = jnp.where(s == 0, EPT0 - REM, EPTN - REM)
        for q in range(REM // 16):
            sv = src_buf[pl.ds(rbase + 16 * q, 16)]
            gidx_r[pl.ds(16 * q, 16)] = sv * RPN + base_g
            didx_r[pl.ds(16 * q, 16)] = dst_buf[pl.ds(rbase + 16 * q, 16)]
        pltpu.async_copy(h_ref.at[gidx_r], rows_r, sem).wait()
        pltpu.sync_copy(rows_r, acc.at[didx_r], add=True)
        if cnt_cond is not None:
            @pl.when(cnt_cond)
            def _():
                pltpu.sync_copy(ones_buf.at[pl.ds(0, REM)],
                                cnt_acc.at[didx_r], add=True)
        plsc.subcore_barrier()

        for half in (0, 1):
            ch = (NCHUNK // 2) * half + (p % (NCHUNK // 2))

            @pl.when((core == half) & (s < NT - 1))
            def _(ch=ch):
                pltpu.sync_copy(
                    acc.at[pl.ds(row0, CROWS)],
                    sum_ref.at[pl.ds(row0, CROWS), pl.ds(ch * CW, CW)])

            @pl.when((core == half) & (s == NT - 1))
            def _(ch=ch):
                pltpu.sync_copy(
                    acc.at[pl.ds(row0, CROWS_L)],
                    sum_ref.at[pl.ds(row0, CROWS_L), pl.ds(ch * CW, CW)])

        if cnt_cond is not None:
            @pl.when(cnt_cond & (s < NT - 1))
            def _():
                pltpu.sync_copy(cnt_acc.at[pl.ds(row0, CROWS)],
                                cnt_out.at[pl.ds(row0, CROWS)])

            @pl.when(cnt_cond & (s == NT - 1))
            def _():
                pltpu.sync_copy(cnt_acc.at[pl.ds(row0, CROWS_L)],
                                cnt_out.at[pl.ds(row0, CROWS_L)])
        plsc.subcore_barrier()


def _aggregate(hu2, hi2, ei_rates, ei_similar):
    z2d = jnp.zeros((ZROWS, CW), jnp.float32)
    z1d = jnp.zeros((CROWS,), jnp.float32)
    o1d = jnp.ones((BATCH,), jnp.float32)
    mesh = plsc.VectorSubcoreMesh(core_axis_name="c", subcore_axis_name="s")
    scratch = [
        pltpu.VMEM((EPT0,), jnp.int32),
        pltpu.VMEM((EPT0,), jnp.int32),
        pltpu.VMEM((KINF, BATCH, CW), jnp.float32),
    ] + [pltpu.VMEM((BATCH,), jnp.int32) for _ in range(2 * KINF)] + [
        pltpu.VMEM((REM,), jnp.int32),
        pltpu.VMEM((REM,), jnp.int32),
        pltpu.VMEM((REM, CW), jnp.float32),
        pltpu.VMEM((BATCH,), jnp.float32),
        pltpu.VMEM((ZROWS, CW), jnp.float32),
        pltpu.VMEM((CROWS,), jnp.float32),
        pltpu.VMEM_SHARED((M, CW), jnp.float32),
        pltpu.VMEM_SHARED((M,), jnp.float32),
        pltpu.SemaphoreType.DMA,
    ]
    out_type = [
        jax.ShapeDtypeStruct((M2, H), jnp.float32),
        jax.ShapeDtypeStruct((M2, H), jnp.float32),
        jax.ShapeDtypeStruct((M2,), jnp.float32),
        jax.ShapeDtypeStruct((M2,), jnp.float32),
    ]
    f = pl.kernel(_agg_body, mesh=mesh, out_type=out_type,
                  scratch_types=scratch,
                  compiler_params=pltpu.CompilerParams(
                      use_tc_tiling_on_sc=False))
    return f(hu2, hi2, ei_rates.reshape(2 * E), ei_similar.reshape(2 * E),
             z2d, z1d, o1d)


def _cnt_col(c_ref):
    slab = c_ref[...]
    ra = lax.broadcasted_iota(jnp.int32, (BM, 16), 0) // 128
    ca = lax.broadcasted_iota(jnp.int32, (BM, 16), 1)
    ea = (ra == ca).astype(jnp.float32)
    t = jnp.dot(ea, slab, preferred_element_type=jnp.float32)
    rb = lax.broadcasted_iota(jnp.int32, (BM, H), 0) % 128
    cb = lax.broadcasted_iota(jnp.int32, (BM, H), 1)
    sel = (rb == cb).astype(jnp.float32)
    return jnp.sum(t * sel, axis=1, keepdims=True)


def _comb_body(s1_ref, c1_ref, s2_ref, c2_ref, h_ref, wr1_ref, wr2_ref,
               a_ref, wc1_ref, bc1_ref, wc2_ref, bc2_ref, o_ref):
    acc1 = jnp.dot(s1_ref[...], wr1_ref[...],
                   preferred_element_type=jnp.float32)
    acc2 = jnp.dot(s2_ref[...], wr2_ref[...],
                   preferred_element_type=jnp.float32)
    cnt1 = _cnt_col(c1_ref)
    cnt2 = _cnt_col(c2_ref)
    msg1 = jnp.where(cnt1 > 0, acc1 / jnp.maximum(cnt1, 1.0), 0.0)
    msg2 = jnp.where(cnt2 > 0, acc2 / jnp.maximum(cnt2, 1.0), 0.0)
    a1 = jnp.dot(msg1, a_ref[...], preferred_element_type=jnp.float32)
    a2 = jnp.dot(msg2, a_ref[...], preferred_element_type=jnp.float32)
    mx = jnp.maximum(a1, a2)
    e1 = jnp.exp(a1 - mx)
    e2 = jnp.exp(a2 - mx)
    inv = 1.0 / (e1 + e2)
    hcomb = msg1 * (e1 * inv) + msg2 * (e2 * inv) + h_ref[...]
    t1 = _gelu(
        jnp.dot(hcomb, wc1_ref[...], preferred_element_type=jnp.float32)
        + bc1_ref[...]
    )
    o_ref[...] = (
        jnp.dot(t1, wc2_ref[...], preferred_element_type=jnp.float32)
        + bc2_ref[...]
    )


def _combine(sum1, cnt1, sum2, cnt2, h_item, wr1, wr2, a_item, wc1, bc1,
             wc2, bc2):
    return pl.pallas_call(
        _comb_body,
        grid=(M2 // BM,),
        in_specs=[
            pl.BlockSpec((BM, H), lambda i: (i, 0)),
            pl.BlockSpec((BM // H, H), lambda i: (i, 0)),
            pl.BlockSpec((BM, H), lambda i: (i, 0)),
            pl.BlockSpec((BM // H, H), lambda i: (i, 0)),
            pl.BlockSpec((BM, H), lambda i: (i, 0)),
            pl.BlockSpec((H, H), lambda i: (0, 0)),
            pl.BlockSpec((H, H), lambda i: (0, 0)),
            pl.BlockSpec((H, 1), lambda i: (0, 0)),
            pl.BlockSpec((H, H), lambda i: (0, 0)),
            pl.BlockSpec((1, H), lambda i: (0, 0)),
            pl.BlockSpec((H, OUT), lambda i: (0, 0)),
            pl.BlockSpec((1, OUT), lambda i: (0, 0)),
        ],
        out_specs=pl.BlockSpec((BM, OUT), lambda i: (i, 0)),
        out_shape=jax.ShapeDtypeStruct((M2, OUT), jnp.float32),
    )(sum1, cnt1.reshape(M2 // H, H), sum2, cnt2.reshape(M2 // H, H),
      h_item, wr1, wr2, a_item, wc1, bc1.reshape(1, H), wc2,
      bc2.reshape(1, OUT))


def kernel(x_user, x_item, ei_rates, ei_rated_by, ei_similar,
           Wu1, bu1, Wu2, bu2, Wi1, bi1, Wi2, bi2,
           Wr_rates, Wr_ratedby, Wr_similar, a_item,
           Wc1, bc1, Wc2, bc2):
    del ei_rated_by, Wr_ratedby
    h_user = _encode(x_user, Wu1, bu1, Wu2, bu2)
    h_item = _encode(x_item, Wi1, bi1, Wi2, bi2)
    sum1, sum2, cnt1, cnt2 = _aggregate(
        h_user.reshape(M2 * RPN, CW), h_item.reshape(M2 * RPN, CW),
        ei_rates, ei_similar)
    logits = _combine(sum1, cnt1, sum2, cnt2, h_item,
                      Wr_rates, Wr_similar, a_item, Wc1, bc1, Wc2, bc2)
    return logits[:M]


# speedup vs baseline: 6.1900x; 1.3220x over previous
import jax
import jax.numpy as jnp
from jax import lax
from jax.experimental import pallas as pl
from jax.experimental.pallas import tpu as pltpu
from jax.experimental.pallas import tpu_sc as plsc

M = 50000
M2 = 51200
E = 400000
D = 128
H = 128
OUT = 64
NCHUNK = 8
CW = 16
RPN = H // CW
NT = 16
BATCH = 128
KINF = 8
EPTN = 24992
EPT0 = EPTN + BATCH
NB = 195
NGROUP = 24
TAIL = NB - NGROUP * KINF
REM = 32
CROWS = 3128
CROWS_L = M - (NT - 1) * CROWS
ZROWS = 400
ZSPLIT = (400,) * 7 + (328,)
ZSPLIT_L = (400,) * 7 + (280,)
BM = 2048


def _gelu(x):
    return 0.5 * x * (1.0 + lax.erf(x * 0.7071067811865476))


def _enc_body(x_ref, w1_ref, b1_ref, w2_ref, b2_ref, o_ref):
    h1 = _gelu(
        jnp.dot(x_ref[...], w1_ref[...], preferred_element_type=jnp.float32)
        + b1_ref[...]
    )
    o_ref[...] = (
        jnp.dot(h1, w2_ref[...], preferred_element_type=jnp.float32)
        + b2_ref[...]
    )


def _encode(x, w1, b1, w2, b2):
    return pl.pallas_call(
        _enc_body,
        grid=(M2 // BM,),
        in_specs=[
            pl.BlockSpec((BM, D), lambda i: (i, 0)),
            pl.BlockSpec((D, H), lambda i: (0, 0)),
            pl.BlockSpec((1, H), lambda i: (0, 0)),
            pl.BlockSpec((H, H), lambda i: (0, 0)),
            pl.BlockSpec((1, H), lambda i: (0, 0)),
        ],
        out_specs=pl.BlockSpec((BM, H), lambda i: (i, 0)),
        out_shape=jax.ShapeDtypeStruct((M2, H), jnp.float32),
    )(x, w1, b1.reshape(1, H), w2, b2.reshape(1, H))


def _agg_body(hu_ref, hi_ref, eir_ref, eis_ref, z2d_ref, z1d_ref, o1d_ref,
              sum1_ref, sum2_ref, cnt1_ref, cnt2_ref,
              src_buf, dst_buf, rows,
              gidx0, gidx1, gidx2, gidx3, gidx4, gidx5, gidx6, gidx7,
              didx0, didx1, didx2, didx3, didx4, didx5, didx6, didx7,
              gidx_r, didx_r, rows_r, ones_buf, zero_buf, zc_buf,
              acc, cnt_acc, sem):
    gidx = [gidx0, gidx1, gidx2, gidx3, gidx4, gidx5, gidx6, gidx7]
    didx = [didx0, didx1, didx2, didx3, didx4, didx5, didx6, didx7]
    core = lax.axis_index("c")
    s = lax.axis_index("s")
    row0 = s * CROWS
    e0 = jnp.where(s == 0, 0, BATCH + s * EPTN)

    pltpu.sync_copy(z2d_ref, zero_buf)
    pltpu.sync_copy(z1d_ref, zc_buf)
    pltpu.sync_copy(o1d_ref, ones_buf)

    def build_idx(gi_ref, di_ref, base, base_g):
        for q in range(BATCH // 16):
            sv = src_buf[pl.ds(base + 16 * q, 16)]
            gi_ref[pl.ds(16 * q, 16)] = sv * RPN + base_g
            di_ref[pl.ds(16 * q, 16)] = dst_buf[pl.ds(base + 16 * q, 16)]

    for p in range(2 * (NCHUNK // 2)):
        rel = p // (NCHUNK // 2)
        ei_ref = eir_ref if rel == 0 else eis_ref
        h_ref = hu_ref if rel == 0 else hi_ref
        sum_ref = sum1_ref if rel == 0 else sum2_ref
        chunk = (NCHUNK // 2) * core + (p % (NCHUNK // 2))
        base_g = chunk
        if p == 0:
            cnt_cond, cnt_out = (core == 0), cnt1_ref
        elif p == NCHUNK // 2:
            cnt_cond, cnt_out = (core == 1), cnt2_ref
        else:
            cnt_cond, cnt_out = None, None

        @pl.when(s < NT - 1)
        def _():
            off = 0
            for zsz in ZSPLIT:
                pltpu.sync_copy(zero_buf.at[pl.ds(0, zsz)],
                                acc.at[pl.ds(row0 + off, zsz)])
                off += zsz

        @pl.when(s == NT - 1)
        def _():
            off = 0
            for zsz in ZSPLIT_L:
                pltpu.sync_copy(zero_buf.at[pl.ds(0, zsz)],
                                acc.at[pl.ds(row0 + off, zsz)])
                off += zsz

        if cnt_cond is not None:
            @pl.when(cnt_cond & (s < NT - 1))
            def _():
                off = 0
                for zsz in ZSPLIT:
                    pltpu.sync_copy(zc_buf.at[pl.ds(0, zsz)],
                                    cnt_acc.at[pl.ds(row0 + off, zsz)])
                    off += zsz

            @pl.when(cnt_cond & (s == NT - 1))
            def _():
                off = 0
                for zsz in ZSPLIT_L:
                    pltpu.sync_copy(zc_buf.at[pl.ds(0, zsz)],
                                    cnt_acc.at[pl.ds(row0 + off, zsz)])
                    off += zsz
        plsc.subcore_barrier()

        if p % (NCHUNK // 2) == 0:
            @pl.when(s == 0)
            def _():
                pltpu.sync_copy(ei_ref.at[pl.ds(0, EPT0)], src_buf)
                pltpu.sync_copy(ei_ref.at[pl.ds(E, EPT0)], dst_buf)

            @pl.when(s > 0)
            def _():
                pltpu.sync_copy(ei_ref.at[pl.ds(e0, EPTN)],
                                src_buf.at[pl.ds(0, EPTN)])
                pltpu.sync_copy(ei_ref.at[pl.ds(E + e0, EPTN)],
                                dst_buf.at[pl.ds(0, EPTN)])

        def fire(k, b):
            build_idx(gidx[k], didx[k], b * BATCH, base_g)
            pltpu.async_copy(h_ref.at[gidx[k]], rows.at[k], sem)

        def drain_scatter(k):
            pltpu.make_async_copy(h_ref.at[gidx[k]], rows.at[k], sem).wait()
            pltpu.sync_copy(rows.at[k], acc.at[didx[k]], add=True)
            if cnt_cond is not None:
                @pl.when(cnt_cond)
                def _():
                    pltpu.sync_copy(ones_buf, cnt_acc.at[didx[k]], add=True)

        for k in range(KINF):
            fire(k, k)

        def group(j, carry):
            for k in range(KINF):
                drain_scatter(k)
                nxt = (j + 1) * KINF + k

                @pl.when(nxt < NB)
                def _(k=k, nxt=nxt):
                    fire(k, nxt)
            return carry

        lax.fori_loop(0, NGROUP, group, 0)
        for k in range(TAIL):
            drain_scatter(k)

        @pl.when(s == 0)
        def _():
            build_idx(gidx[0], didx[0], NGROUP * KINF * BATCH, base_g)
            pltpu.async_copy(h_ref.at[gidx[0]], rows.at[0], sem).wait()
            pltpu.sync_copy(rows.at[0], acc.at[didx[0]], add=True)
            if cnt_cond is not None:
                @pl.when(cnt_cond)
                def _():
                    pltpu.sync_copy(ones_buf, cnt_acc.at[didx[0]], add=True)

        rbase = jnp.where(s == 0, EPT0 - REM, EPTN - REM)
        for q in range(REM // 16):
            sv = src_buf[pl.ds(rbase + 16 * q, 16)]
            gidx_r[pl.ds(16 * q, 16)] = sv * RPN + base_g
            didx_r[pl.ds(16 * q, 16)] = dst_buf[pl.ds(rbase + 16 * q, 16)]
        pltpu.async_copy(h_ref.at[gidx_r], rows_r, sem).wait()
        pltpu.sync_copy(rows_r, acc.at[didx_r], add=True)
        if cnt_cond is not None:
            @pl.when(cnt_cond)
            def _():
                pltpu.sync_copy(ones_buf.at[pl.ds(0, REM)],
                                cnt_acc.at[didx_r], add=True)
        plsc.subcore_barrier()

        for half in (0, 1):
            ch = (NCHUNK // 2) * half + (p % (NCHUNK // 2))

            @pl.when((core == half) & (s < NT - 1))
            def _(ch=ch):
                pltpu.sync_copy(
                    acc.at[pl.ds(row0, CROWS)],
                    sum_ref.at[pl.ds(row0, CROWS), pl.ds(ch * CW, CW)])

            @pl.when((core == half) & (s == NT - 1))
            def _(ch=ch):
                pltpu.sync_copy(
                    acc.at[pl.ds(row0, CROWS_L)],
                    sum_ref.at[pl.ds(row0, CROWS_L), pl.ds(ch * CW, CW)])

        if cnt_cond is not None:
            @pl.when(cnt_cond & (s < NT - 1))
            def _():
                pltpu.sync_copy(cnt_acc.at[pl.ds(row0, CROWS)],
                                cnt_out.at[pl.ds(row0, CROWS)])

            @pl.when(cnt_cond & (s == NT - 1))
            def _():
                pltpu.sync_copy(cnt_acc.at[pl.ds(row0, CROWS_L)],
                                cnt_out.at[pl.ds(row0, CROWS_L)])
        plsc.subcore_barrier()


def _aggregate(hu2, hi2, ei_rates, ei_similar):
    z2d = jnp.zeros((ZROWS, CW), jnp.float32)
    z1d = jnp.zeros((ZROWS,), jnp.float32)
    o1d = jnp.ones((BATCH,), jnp.float32)
    mesh = plsc.VectorSubcoreMesh(core_axis_name="c", subcore_axis_name="s")
    scratch = [
        pltpu.VMEM((EPT0,), jnp.int32),
        pltpu.VMEM((EPT0,), jnp.int32),
        pltpu.VMEM((KINF, BATCH, CW), jnp.float32),
    ] + [pltpu.VMEM((BATCH,), jnp.int32) for _ in range(2 * KINF)] + [
        pltpu.VMEM((REM,), jnp.int32),
        pltpu.VMEM((REM,), jnp.int32),
        pltpu.VMEM((REM, CW), jnp.float32),
        pltpu.VMEM((BATCH,), jnp.float32),
        pltpu.VMEM((ZROWS, CW), jnp.float32),
        pltpu.VMEM((ZROWS,), jnp.float32),
        pltpu.VMEM_SHARED((M, CW), jnp.float32),
        pltpu.VMEM_SHARED((M,), jnp.float32),
        pltpu.SemaphoreType.DMA,
    ]
    out_type = [
        jax.ShapeDtypeStruct((M2, H), jnp.float32),
        jax.ShapeDtypeStruct((M2, H), jnp.float32),
        jax.ShapeDtypeStruct((M2,), jnp.float32),
        jax.ShapeDtypeStruct((M2,), jnp.float32),
    ]
    f = pl.kernel(_agg_body, mesh=mesh, out_type=out_type,
                  scratch_types=scratch,
                  compiler_params=pltpu.CompilerParams(
                      use_tc_tiling_on_sc=False))
    return f(hu2, hi2, ei_rates.reshape(2 * E), ei_similar.reshape(2 * E),
             z2d, z1d, o1d)


def _cnt_col(c_ref):
    slab = c_ref[...]
    ra = lax.broadcasted_iota(jnp.int32, (BM, 16), 0) // 128
    ca = lax.broadcasted_iota(jnp.int32, (BM, 16), 1)
    ea = (ra == ca).astype(jnp.float32)
    t = jnp.dot(ea, slab, preferred_element_type=jnp.float32)
    rb = lax.broadcasted_iota(jnp.int32, (BM, H), 0) % 128
    cb = lax.broadcasted_iota(jnp.int32, (BM, H), 1)
    sel = (rb == cb).astype(jnp.float32)
    return jnp.sum(t * sel, axis=1, keepdims=True)


def _comb_body(s1_ref, c1_ref, s2_ref, c2_ref, h_ref, wr1_ref, wr2_ref,
               a_ref, wc1_ref, bc1_ref, wc2_ref, bc2_ref, o_ref):
    acc1 = jnp.dot(s1_ref[...], wr1_ref[...],
                   preferred_element_type=jnp.float32)
    acc2 = jnp.dot(s2_ref[...], wr2_ref[...],
                   preferred_element_type=jnp.float32)
    cnt1 = _cnt_col(c1_ref)
    cnt2 = _cnt_col(c2_ref)
    msg1 = jnp.where(cnt1 > 0, acc1 / jnp.maximum(cnt1, 1.0), 0.0)
    msg2 = jnp.where(cnt2 > 0, acc2 / jnp.maximum(cnt2, 1.0), 0.0)
    a1 = jnp.dot(msg1, a_ref[...], preferred_element_type=jnp.float32)
    a2 = jnp.dot(msg2, a_ref[...], preferred_element_type=jnp.float32)
    mx = jnp.maximum(a1, a2)
    e1 = jnp.exp(a1 - mx)
    e2 = jnp.exp(a2 - mx)
    inv = 1.0 / (e1 + e2)
    hcomb = msg1 * (e1 * inv) + msg2 * (e2 * inv) + h_ref[...]
    t1 = _gelu(
        jnp.dot(hcomb, wc1_ref[...], preferred_element_type=jnp.float32)
        + bc1_ref[...]
    )
    o_ref[...] = (
        jnp.dot(t1, wc2_ref[...], preferred_element_type=jnp.float32)
        + bc2_ref[...]
    )


def _combine(sum1, cnt1, sum2, cnt2, h_item, wr1, wr2, a_item, wc1, bc1,
             wc2, bc2):
    return pl.pallas_call(
        _comb_body,
        grid=(M2 // BM,),
        in_specs=[
            pl.BlockSpec((BM, H), lambda i: (i, 0)),
            pl.BlockSpec((BM // H, H), lambda i: (i, 0)),
            pl.BlockSpec((BM, H), lambda i: (i, 0)),
            pl.BlockSpec((BM // H, H), lambda i: (i, 0)),
            pl.BlockSpec((BM, H), lambda i: (i, 0)),
            pl.BlockSpec((H, H), lambda i: (0, 0)),
            pl.BlockSpec((H, H), lambda i: (0, 0)),
            pl.BlockSpec((H, 1), lambda i: (0, 0)),
            pl.BlockSpec((H, H), lambda i: (0, 0)),
            pl.BlockSpec((1, H), lambda i: (0, 0)),
            pl.BlockSpec((H, OUT), lambda i: (0, 0)),
            pl.BlockSpec((1, OUT), lambda i: (0, 0)),
        ],
        out_specs=pl.BlockSpec((BM, OUT), lambda i: (i, 0)),
        out_shape=jax.ShapeDtypeStruct((M2, OUT), jnp.float32),
    )(sum1, cnt1.reshape(M2 // H, H), sum2, cnt2.reshape(M2 // H, H),
      h_item, wr1, wr2, a_item, wc1, bc1.reshape(1, H), wc2,
      bc2.reshape(1, OUT))


def kernel(x_user, x_item, ei_rates, ei_rated_by, ei_similar,
           Wu1, bu1, Wu2, bu2, Wi1, bi1, Wi2, bi2,
           Wr_rates, Wr_ratedby, Wr_similar, a_item,
           Wc1, bc1, Wc2, bc2):
    del ei_rated_by, Wr_ratedby
    h_user = _encode(x_user, Wu1, bu1, Wu2, bu2)
    h_item = _encode(x_item, Wi1, bi1, Wi2, bi2)
    sum1, sum2, cnt1, cnt2 = _aggregate(
        h_user.reshape(M2 * RPN, CW), h_item.reshape(M2 * RPN, CW),
        ei_rates, ei_similar)
    logits = _combine(sum1, cnt1, sum2, cnt2, h_item,
                      Wr_rates, Wr_similar, a_item, Wc1, bc1, Wc2, bc2)
    return logits[:M]


# speedup vs baseline: 8.4642x; 1.3674x over previous
import jax
import jax.numpy as jnp
from jax import lax
from jax.experimental import pallas as pl
from jax.experimental.pallas import tpu as pltpu
from jax.experimental.pallas import tpu_sc as plsc

M = 50000
M2 = 51200
E = 400000
D = 128
H = 128
OUT = 64
NCHUNK = 4
CW = 32
RPN = H // CW
NT = 16
BATCH = 128
KINF = 4
EPTN = 24992
EPT0 = EPTN + BATCH
WB = 8
WE = WB * BATCH
NWIN = 24
NB_MAIN = NWIN * WB
TAILB = 3
REM = 32
TAIL_N = TAILB * BATCH + REM
TAIL_0 = TAIL_N + BATCH
CROWS = 3128
CROWS_L = M - (NT - 1) * CROWS
ZR2 = 100
ZSPLIT = (ZR2,) * 31 + (28,)
ZSPLIT_L = (ZR2,) * 30 + (80,)
CZROWS = 400
CSPLIT = (CZROWS,) * 7 + (328,)
CSPLIT_L = (CZROWS,) * 7 + (280,)
BM = 2048


def _gelu(x):
    return 0.5 * x * (1.0 + lax.erf(x * 0.7071067811865476))


def _enc_body(x_ref, w1_ref, b1_ref, w2_ref, b2_ref, o_ref):
    h1 = _gelu(
        jnp.dot(x_ref[...], w1_ref[...], preferred_element_type=jnp.float32)
        + b1_ref[...]
    )
    o_ref[...] = (
        jnp.dot(h1, w2_ref[...], preferred_element_type=jnp.float32)
        + b2_ref[...]
    )


def _encode(x, w1, b1, w2, b2):
    return pl.pallas_call(
        _enc_body,
        grid=(M2 // BM,),
        in_specs=[
            pl.BlockSpec((BM, D), lambda i: (i, 0)),
            pl.BlockSpec((D, H), lambda i: (0, 0)),
            pl.BlockSpec((1, H), lambda i: (0, 0)),
            pl.BlockSpec((H, H), lambda i: (0, 0)),
            pl.BlockSpec((1, H), lambda i: (0, 0)),
        ],
        out_specs=pl.BlockSpec((BM, H), lambda i: (i, 0)),
        out_shape=jax.ShapeDtypeStruct((M2, H), jnp.float32),
    )(x, w1, b1.reshape(1, H), w2, b2.reshape(1, H))


def _agg_body(hu_ref, hi_ref, eir_ref, eis_ref, z2d_ref, z1d_ref, o1d_ref,
              sum1_ref, sum2_ref, cnt1_ref, cnt2_ref,
              wsrc0, wdst0, wsrc1, wdst1, rows,
              gidx0, gidx1, gidx2, gidx3,
              didx0, didx1, didx2, didx3,
              gidx_r, didx_r, rows_r, ones_buf, zero_buf, zc_buf,
              acc, cnt_acc, sem, sem2):
    gidx = [gidx0, gidx1, gidx2, gidx3]
    didx = [didx0, didx1, didx2, didx3]
    wsrc = [wsrc0, wsrc1]
    wdst = [wdst0, wdst1]
    core = lax.axis_index("c")
    s = lax.axis_index("s")
    row0 = s * CROWS
    e0 = jnp.where(s == 0, 0, BATCH + s * EPTN)

    pltpu.sync_copy(z2d_ref, zero_buf)
    pltpu.sync_copy(z1d_ref, zc_buf)
    pltpu.sync_copy(o1d_ref, ones_buf)

    for p in range(2 * (NCHUNK // 2)):
        rel = p // (NCHUNK // 2)
        ei_ref = eir_ref if rel == 0 else eis_ref
        h_ref = hu_ref if rel == 0 else hi_ref
        sum_ref = sum1_ref if rel == 0 else sum2_ref
        chunk = (NCHUNK // 2) * core + (p % (NCHUNK // 2))
        base_g = chunk
        if p == 0:
            cnt_cond, cnt_out = (core == 0), cnt1_ref
        elif p == NCHUNK // 2:
            cnt_cond, cnt_out = (core == 1), cnt2_ref
        else:
            cnt_cond, cnt_out = None, None

        @pl.when(s < NT - 1)
        def _():
            off = 0
            for zsz in ZSPLIT:
                pltpu.sync_copy(zero_buf.at[pl.ds(0, zsz)],
                                acc.at[pl.ds(row0 + off, zsz)])
                off += zsz

        @pl.when(s == NT - 1)
        def _():
            off = 0
            for zsz in ZSPLIT_L:
                pltpu.sync_copy(zero_buf.at[pl.ds(0, zsz)],
                                acc.at[pl.ds(row0 + off, zsz)])
                off += zsz

        if cnt_cond is not None:
            @pl.when(cnt_cond & (s < NT - 1))
            def _():
                off = 0
                for zsz in CSPLIT:
                    pltpu.sync_copy(zc_buf.at[pl.ds(0, zsz)],
                                    cnt_acc.at[pl.ds(row0 + off, zsz)])
                    off += zsz

            @pl.when(cnt_cond & (s == NT - 1))
            def _():
                off = 0
                for zsz in CSPLIT_L:
                    pltpu.sync_copy(zc_buf.at[pl.ds(0, zsz)],
                                    cnt_acc.at[pl.ds(row0 + off, zsz)])
                    off += zsz
        plsc.subcore_barrier()

        def stage_window_async(w, parity):
            pltpu.async_copy(ei_ref.at[pl.ds(e0 + w * WE, WE)],
                             wsrc[parity], sem2)
            pltpu.async_copy(ei_ref.at[pl.ds(E + e0 + w * WE, WE)],
                             wdst[parity], sem2)

        def wait_window(parity):
            pltpu.make_async_copy(ei_ref.at[pl.ds(0, WE)],
                                  wsrc[parity], sem2).wait()
            pltpu.make_async_copy(ei_ref.at[pl.ds(0, WE)],
                                  wdst[parity], sem2).wait()

        def build_idx(gi_ref, di_ref, srcb, dstb, loc):
            for q in range(BATCH // 16):
                sv = srcb[pl.ds(loc + 16 * q, 16)]
                gi_ref[pl.ds(16 * q, 16)] = sv * RPN + base_g
                di_ref[pl.ds(16 * q, 16)] = dstb[pl.ds(loc + 16 * q, 16)]

        def fire(k, parity, loc):
            build_idx(gidx[k], didx[k], wsrc[parity], wdst[parity], loc)
            pltpu.async_copy(h_ref.at[gidx[k]], rows.at[k], sem)

        def drain_scatter(k):
            pltpu.make_async_copy(h_ref.at[gidx[k]], rows.at[k], sem).wait()
            pltpu.sync_copy(rows.at[k], acc.at[didx[k]], add=True)
            if cnt_cond is not None:
                @pl.when(cnt_cond)
                def _():
                    pltpu.sync_copy(ones_buf, cnt_acc.at[didx[k]], add=True)

        stage_window_async(0, 0)
        wait_window(0)
        stage_window_async(1, 1)
        for k in range(KINF):
            fire(k, 0, k * BATCH)

        def wpair(j, carry):
            for parity in (0, 1):
                w = 2 * j + parity
                for k in range(KINF):
                    drain_scatter(k)
                    fire(k, parity, (KINF + k) * BATCH)
                @pl.when(w + 1 < NWIN)
                def _(parity=parity):
                    wait_window(1 - parity)

                @pl.when(w + 2 < NWIN)
                def _(w=w, parity=parity):
                    stage_window_async(w + 2, parity)
                for k in range(KINF):
                    drain_scatter(k)

                    @pl.when(w * WB + WB + k < NB_MAIN)
                    def _(k=k, parity=parity):
                        fire(k, 1 - parity, k * BATCH)
            return carry

        lax.fori_loop(0, NWIN // 2, wpair, 0)

        @pl.when(s == 0)
        def _():
            pltpu.sync_copy(ei_ref.at[pl.ds(e0 + NWIN * WE, TAIL_0)],
                            wsrc0.at[pl.ds(0, TAIL_0)])
            pltpu.sync_copy(ei_ref.at[pl.ds(E + e0 + NWIN * WE, TAIL_0)],
                            wdst0.at[pl.ds(0, TAIL_0)])

        @pl.when(s > 0)
        def _():
            pltpu.sync_copy(ei_ref.at[pl.ds(e0 + NWIN * WE, TAIL_N)],
                            wsrc0.at[pl.ds(0, TAIL_N)])
            pltpu.sync_copy(ei_ref.at[pl.ds(E + e0 + NWIN * WE, TAIL_N)],
                            wdst0.at[pl.ds(0, TAIL_N)])

        for k in range(TAILB):
            fire(k, 0, k * BATCH)

        @pl.when(s == 0)
        def _():
            fire(TAILB, 0, TAILB * BATCH)
        for k in range(TAILB):
            drain_scatter(k)

        @pl.when(s == 0)
        def _():
            drain_scatter(TAILB)

        rloc = jnp.where(s == 0, TAIL_0 - REM, TAIL_N - REM)
        for q in range(REM // 16):
            sv = wsrc0[pl.ds(rloc + 16 * q, 16)]
            gidx_r[pl.ds(16 * q, 16)] = sv * RPN + base_g
            didx_r[pl.ds(16 * q, 16)] = wdst0[pl.ds(rloc + 16 * q, 16)]
        pltpu.async_copy(h_ref.at[gidx_r], rows_r, sem).wait()
        pltpu.sync_copy(rows_r, acc.at[didx_r], add=True)
        if cnt_cond is not None:
            @pl.when(cnt_cond)
            def _():
                pltpu.sync_copy(ones_buf.at[pl.ds(0, REM)],
                                cnt_acc.at[didx_r], add=True)
        plsc.subcore_barrier()

        for half in (0, 1):
            ch = (NCHUNK // 2) * half + (p % (NCHUNK // 2))

            @pl.when((core == half) & (s < NT - 1))
            def _(ch=ch):
                pltpu.sync_copy(
                    acc.at[pl.ds(row0, CROWS)],
                    sum_ref.at[pl.ds(row0, CROWS), pl.ds(ch * CW, CW)])

            @pl.when((core == half) & (s == NT - 1))
            def _(ch=ch):
                pltpu.sync_copy(
                    acc.at[pl.ds(row0, CROWS_L)],
                    sum_ref.at[pl.ds(row0, CROWS_L), pl.ds(ch * CW, CW)])

        if cnt_cond is not None:
            @pl.when(cnt_cond & (s < NT - 1))
            def _():
                pltpu.sync_copy(cnt_acc.at[pl.ds(row0, CROWS)],
                                cnt_out.at[pl.ds(row0, CROWS)])

            @pl.when(cnt_cond & (s == NT - 1))
            def _():
                pltpu.sync_copy(cnt_acc.at[pl.ds(row0, CROWS_L)],
                                cnt_out.at[pl.ds(row0, CROWS_L)])
        plsc.subcore_barrier()


def _aggregate(hu2, hi2, ei_rates, ei_similar):
    z2d = jnp.zeros((ZR2, CW), jnp.float32)
    z1d = jnp.zeros((CZROWS,), jnp.float32)
    o1d = jnp.ones((BATCH,), jnp.float32)
    mesh = plsc.VectorSubcoreMesh(core_axis_name="c", subcore_axis_name="s")
    scratch = [
        pltpu.VMEM((WE,), jnp.int32),
        pltpu.VMEM((WE,), jnp.int32),
        pltpu.VMEM((WE,), jnp.int32),
        pltpu.VMEM((WE,), jnp.int32),
        pltpu.VMEM((KINF, BATCH, CW), jnp.float32),
    ] + [pltpu.VMEM((BATCH,), jnp.int32) for _ in range(2 * KINF)] + [
        pltpu.VMEM((REM,), jnp.int32),
        pltpu.VMEM((REM,), jnp.int32),
        pltpu.VMEM((REM, CW), jnp.float32),
        pltpu.VMEM((BATCH,), jnp.float32),
        pltpu.VMEM((ZR2, CW), jnp.float32),
        pltpu.VMEM((CZROWS,), jnp.float32),
        pltpu.VMEM_SHARED((M, CW), jnp.float32),
        pltpu.VMEM_SHARED((M,), jnp.float32),
        pltpu.SemaphoreType.DMA,
        pltpu.SemaphoreType.DMA,
    ]
    out_type = [
        jax.ShapeDtypeStruct((M2, H), jnp.float32),
        jax.ShapeDtypeStruct((M2, H), jnp.float32),
        jax.ShapeDtypeStruct((M2,), jnp.float32),
        jax.ShapeDtypeStruct((M2,), jnp.float32),
    ]
    f = pl.kernel(_agg_body, mesh=mesh, out_type=out_type,
                  scratch_types=scratch,
                  compiler_params=pltpu.CompilerParams(
                      use_tc_tiling_on_sc=False))
    return f(hu2, hi2, ei_rates.reshape(2 * E), ei_similar.reshape(2 * E),
             z2d, z1d, o1d)


def _cnt_col(c_ref):
    slab = c_ref[...]
    ra = lax.broadcasted_iota(jnp.int32, (BM, 16), 0) // 128
    ca = lax.broadcasted_iota(jnp.int32, (BM, 16), 1)
    ea = (ra == ca).astype(jnp.float32)
    t = jnp.dot(ea, slab, preferred_element_type=jnp.float32)
    rb = lax.broadcasted_iota(jnp.int32, (BM, H), 0) % 128
    cb = lax.broadcasted_iota(jnp.int32, (BM, H), 1)
    sel = (rb == cb).astype(jnp.float32)
    return jnp.sum(t * sel, axis=1, keepdims=True)


def _comb_body(s1_ref, c1_ref, s2_ref, c2_ref, h_ref, wr1_ref, wr2_ref,
               a_ref, wc1_ref, bc1_ref, wc2_ref, bc2_ref, o_ref):
    acc1 = jnp.dot(s1_ref[...], wr1_ref[...],
                   preferred_element_type=jnp.float32)
    acc2 = jnp.dot(s2_ref[...], wr2_ref[...],
                   preferred_element_type=jnp.float32)
    cnt1 = _cnt_col(c1_ref)
    cnt2 = _cnt_col(c2_ref)
    msg1 = jnp.where(cnt1 > 0, acc1 / jnp.maximum(cnt1, 1.0), 0.0)
    msg2 = jnp.where(cnt2 > 0, acc2 / jnp.maximum(cnt2, 1.0), 0.0)
    a1 = jnp.dot(msg1, a_ref[...], preferred_element_type=jnp.float32)
    a2 = jnp.dot(msg2, a_ref[...], preferred_element_type=jnp.float32)
    mx = jnp.maximum(a1, a2)
    e1 = jnp.exp(a1 - mx)
    e2 = jnp.exp(a2 - mx)
    inv = 1.0 / (e1 + e2)
    hcomb = msg1 * (e1 * inv) + msg2 * (e2 * inv) + h_ref[...]
    t1 = _gelu(
        jnp.dot(hcomb, wc1_ref[...], preferred_element_type=jnp.float32)
        + bc1_ref[...]
    )
    o_ref[...] = (
        jnp.dot(t1, wc2_ref[...], preferred_element_type=jnp.float32)
        + bc2_ref[...]
    )


def _combine(sum1, cnt1, sum2, cnt2, h_item, wr1, wr2, a_item, wc1, bc1,
             wc2, bc2):
    return pl.pallas_call(
        _comb_body,
        grid=(M2 // BM,),
        in_specs=[
            pl.BlockSpec((BM, H), lambda i: (i, 0)),
            pl.BlockSpec((BM // H, H), lambda i: (i, 0)),
            pl.BlockSpec((BM, H), lambda i: (i, 0)),
            pl.BlockSpec((BM // H, H), lambda i: (i, 0)),
            pl.BlockSpec((BM, H), lambda i: (i, 0)),
            pl.BlockSpec((H, H), lambda i: (0, 0)),
            pl.BlockSpec((H, H), lambda i: (0, 0)),
            pl.BlockSpec((H, 1), lambda i: (0, 0)),
            pl.BlockSpec((H, H), lambda i: (0, 0)),
            pl.BlockSpec((1, H), lambda i: (0, 0)),
            pl.BlockSpec((H, OUT), lambda i: (0, 0)),
            pl.BlockSpec((1, OUT), lambda i: (0, 0)),
        ],
        out_specs=pl.BlockSpec((BM, OUT), lambda i: (i, 0)),
        out_shape=jax.ShapeDtypeStruct((M2, OUT), jnp.float32),
    )(sum1, cnt1.reshape(M2 // H, H), sum2, cnt2.reshape(M2 // H, H),
      h_item, wr1, wr2, a_item, wc1, bc1.reshape(1, H), wc2,
      bc2.reshape(1, OUT))


def kernel(x_user, x_item, ei_rates, ei_rated_by, ei_similar,
           Wu1, bu1, Wu2, bu2, Wi1, bi1, Wi2, bi2,
           Wr_rates, Wr_ratedby, Wr_similar, a_item,
           Wc1, bc1, Wc2, bc2):
    del ei_rated_by, Wr_ratedby
    h_user = _encode(x_user, Wu1, bu1, Wu2, bu2)
    h_item = _encode(x_item, Wi1, bi1, Wi2, bi2)
    sum1, sum2, cnt1, cnt2 = _aggregate(
        h_user.reshape(M2 * RPN, CW), h_item.reshape(M2 * RPN, CW),
        ei_rates, ei_similar)
    logits = _combine(sum1, cnt1, sum2, cnt2, h_item,
                      Wr_rates, Wr_similar, a_item, Wc1, bc1, Wc2, bc2)
    return logits[:M]


# speedup vs baseline: 9.1181x; 1.0773x over previous
import jax
import jax.numpy as jnp
from jax import lax
from jax.experimental import pallas as pl
from jax.experimental.pallas import tpu as pltpu
from jax.experimental.pallas import tpu_sc as plsc

M = 50000
M2 = 51200
E = 400000
D = 128
H = 128
OUT = 64
NCHUNK = 4
CW = 32
RPN = H // CW
NT = 16
BATCH = 128
KINF = 4
EPTN = 24992
EPT0 = EPTN + BATCH
WB = 8
WE = WB * BATCH
NWIN = 24
NB_MAIN = NWIN * WB
TAILB = 3
REM = 32
TAIL_N = TAILB * BATCH + REM
TAIL_0 = TAIL_N + BATCH
CROWS = 3128
CROWS_L = M - (NT - 1) * CROWS
ZR2 = 100
ZSPLIT = (ZR2,) * 31 + (28,)
ZSPLIT_L = (ZR2,) * 30 + (80,)
CZROWS = 400
CSPLIT = (CZROWS,) * 7 + (328,)
CSPLIT_L = (CZROWS,) * 7 + (280,)
BM = 2048


def _gelu(x):
    return 0.5 * x * (1.0 + lax.erf(x * 0.7071067811865476))


def _enc_body(x_ref, w1_ref, b1_ref, w2_ref, b2_ref, o_ref):
    h1 = _gelu(
        jnp.dot(x_ref[...], w1_ref[...], preferred_element_type=jnp.float32)
        + b1_ref[...]
    )
    o_ref[...] = (
        jnp.dot(h1, w2_ref[...], preferred_element_type=jnp.float32)
        + b2_ref[...]
    )


def _encode(x, w1, b1, w2, b2):
    return pl.pallas_call(
        _enc_body,
        grid=(M2 // BM,),
        in_specs=[
            pl.BlockSpec((BM, D), lambda i: (i, 0)),
            pl.BlockSpec((D, H), lambda i: (0, 0)),
            pl.BlockSpec((1, H), lambda i: (0, 0)),
            pl.BlockSpec((H, H), lambda i: (0, 0)),
            pl.BlockSpec((1, H), lambda i: (0, 0)),
        ],
        out_specs=pl.BlockSpec((BM, H), lambda i: (i, 0)),
        out_shape=jax.ShapeDtypeStruct((M2, H), jnp.float32),
    )(x, w1, b1.reshape(1, H), w2, b2.reshape(1, H))


def _make_agg_body(cnt_core):
    def _agg_body(h_ref, ei_ref, z2d_ref, z1d_ref, o1d_ref,
                  sum_ref, cnt_out,
                  wsrc0, wdst0, wsrc1, wdst1, rows,
                  gidx0, gidx1, gidx2, gidx3,
                  didx0, didx1, didx2, didx3,
                  gidx_r, didx_r, rows_r, ones_buf, zero_buf, zc_buf,
                  acc, cnt_acc, sem, sem2):
        _agg_passes(h_ref, ei_ref, z2d_ref, z1d_ref, o1d_ref,
                    sum_ref, cnt_out,
                    [wsrc0, wsrc1], [wdst0, wdst1], rows,
                    [gidx0, gidx1, gidx2, gidx3],
                    [didx0, didx1, didx2, didx3],
                    gidx_r, didx_r, rows_r, ones_buf, zero_buf, zc_buf,
                    acc, cnt_acc, sem, sem2, cnt_core)
    return _agg_body


def _agg_passes(h_ref, ei_ref, z2d_ref, z1d_ref, o1d_ref, sum_ref, cnt_out,
                wsrc, wdst, rows, gidx, didx,
                gidx_r, didx_r, rows_r, ones_buf, zero_buf, zc_buf,
                acc, cnt_acc, sem, sem2, cnt_core):
    core = lax.axis_index("c")
    s = lax.axis_index("s")
    row0 = s * CROWS
    e0 = jnp.where(s == 0, 0, BATCH + s * EPTN)

    pltpu.sync_copy(z2d_ref, zero_buf)
    pltpu.sync_copy(z1d_ref, zc_buf)
    pltpu.sync_copy(o1d_ref, ones_buf)

    for p in range(NCHUNK // 2):
        chunk = (NCHUNK // 2) * core + p
        base_g = chunk
        if p == 0:
            cnt_cond = core == cnt_core
        else:
            cnt_cond = None

        @pl.when(s < NT - 1)
        def _():
            off = 0
            for zsz in ZSPLIT:
                pltpu.sync_copy(zero_buf.at[pl.ds(0, zsz)],
                                acc.at[pl.ds(row0 + off, zsz)])
                off += zsz

        @pl.when(s == NT - 1)
        def _():
            off = 0
            for zsz in ZSPLIT_L:
                pltpu.sync_copy(zero_buf.at[pl.ds(0, zsz)],
                                acc.at[pl.ds(row0 + off, zsz)])
                off += zsz

        if cnt_cond is not None:
            @pl.when(cnt_cond & (s < NT - 1))
            def _():
                off = 0
                for zsz in CSPLIT:
                    pltpu.sync_copy(zc_buf.at[pl.ds(0, zsz)],
                                    cnt_acc.at[pl.ds(row0 + off, zsz)])
                    off += zsz

            @pl.when(cnt_cond & (s == NT - 1))
            def _():
                off = 0
                for zsz in CSPLIT_L:
                    pltpu.sync_copy(zc_buf.at[pl.ds(0, zsz)],
                                    cnt_acc.at[pl.ds(row0 + off, zsz)])
                    off += zsz
        plsc.subcore_barrier()

        def stage_window_async(w, parity):
            pltpu.async_copy(ei_ref.at[pl.ds(e0 + w * WE, WE)],
                             wsrc[parity], sem2)
            pltpu.async_copy(ei_ref.at[pl.ds(E + e0 + w * WE, WE)],
                             wdst[parity], sem2)

        def wait_window(parity):
            pltpu.make_async_copy(ei_ref.at[pl.ds(0, WE)],
                                  wsrc[parity], sem2).wait()
            pltpu.make_async_copy(ei_ref.at[pl.ds(0, WE)],
                                  wdst[parity], sem2).wait()

        def build_idx(gi_ref, di_ref, srcb, dstb, loc):
            for q in range(BATCH // 16):
                sv = srcb[pl.ds(loc + 16 * q, 16)]
                gi_ref[pl.ds(16 * q, 16)] = sv * RPN + base_g
                di_ref[pl.ds(16 * q, 16)] = dstb[pl.ds(loc + 16 * q, 16)]

        def fire(k, parity, loc):
            build_idx(gidx[k], didx[k], wsrc[parity], wdst[parity], loc)
            pltpu.async_copy(h_ref.at[gidx[k]], rows.at[k], sem)

        def drain_scatter(k):
            pltpu.make_async_copy(h_ref.at[gidx[k]], rows.at[k], sem).wait()
            pltpu.sync_copy(rows.at[k], acc.at[didx[k]], add=True)
            if cnt_cond is not None:
                @pl.when(cnt_cond)
                def _():
                    pltpu.sync_copy(ones_buf, cnt_acc.at[didx[k]], add=True)

        stage_window_async(0, 0)
        wait_window(0)
        stage_window_async(1, 1)
        for k in range(KINF):
            fire(k, 0, k * BATCH)

        def wpair(j, carry):
            for parity in (0, 1):
                w = 2 * j + parity
                for k in range(KINF):
                    drain_scatter(k)
                    fire(k, parity, (KINF + k) * BATCH)
                @pl.when(w + 1 < NWIN)
                def _(parity=parity):
                    wait_window(1 - parity)

                @pl.when(w + 2 < NWIN)
                def _(w=w, parity=parity):
                    stage_window_async(w + 2, parity)
                for k in range(KINF):
                    drain_scatter(k)

                    @pl.when(w * WB + WB + k < NB_MAIN)
                    def _(k=k, parity=parity):
                        fire(k, 1 - parity, k * BATCH)
            return carry

        lax.fori_loop(0, NWIN // 2, wpair, 0)

        @pl.when(s == 0)
        def _():
            pltpu.sync_copy(ei_ref.at[pl.ds(e0 + NWIN * WE, TAIL_0)],
                            wsrc[0].at[pl.ds(0, TAIL_0)])
            pltpu.sync_copy(ei_ref.at[pl.ds(E + e0 + NWIN * WE, TAIL_0)],
                            wdst[0].at[pl.ds(0, TAIL_0)])

        @pl.when(s > 0)
        def _():
            pltpu.sync_copy(ei_ref.at[pl.ds(e0 + NWIN * WE, TAIL_N)],
                            wsrc[0].at[pl.ds(0, TAIL_N)])
            pltpu.sync_copy(ei_ref.at[pl.ds(E + e0 + NWIN * WE, TAIL_N)],
                            wdst[0].at[pl.ds(0, TAIL_N)])

        for k in range(TAILB):
            fire(k, 0, k * BATCH)

        @pl.when(s == 0)
        def _():
            fire(TAILB, 0, TAILB * BATCH)
        for k in range(TAILB):
            drain_scatter(k)

        @pl.when(s == 0)
        def _():
            drain_scatter(TAILB)

        rloc = jnp.where(s == 0, TAIL_0 - REM, TAIL_N - REM)
        for q in range(REM // 16):
            sv = wsrc[0][pl.ds(rloc + 16 * q, 16)]
            gidx_r[pl.ds(16 * q, 16)] = sv * RPN + base_g
            didx_r[pl.ds(16 * q, 16)] = wdst[0][pl.ds(rloc + 16 * q, 16)]
        pltpu.async_copy(h_ref.at[gidx_r], rows_r, sem).wait()
        pltpu.sync_copy(rows_r, acc.at[didx_r], add=True)
        if cnt_cond is not None:
            @pl.when(cnt_cond)
            def _():
                pltpu.sync_copy(ones_buf.at[pl.ds(0, REM)],
                                cnt_acc.at[didx_r], add=True)
        plsc.subcore_barrier()

        for half in (0, 1):
            ch = (NCHUNK // 2) * half + (p % (NCHUNK // 2))

            @pl.when((core == half) & (s < NT - 1))
            def _(ch=ch):
                pltpu.sync_copy(
                    acc.at[pl.ds(row0, CROWS)],
                    sum_ref.at[pl.ds(row0, CROWS), pl.ds(ch * CW, CW)])

            @pl.when((core == half) & (s == NT - 1))
            def _(ch=ch):
                pltpu.sync_copy(
                    acc.at[pl.ds(row0, CROWS_L)],
                    sum_ref.at[pl.ds(row0, CROWS_L), pl.ds(ch * CW, CW)])

        if cnt_cond is not None:
            @pl.when(cnt_cond & (s < NT - 1))
            def _():
                pltpu.sync_copy(cnt_acc.at[pl.ds(row0, CROWS)],
                                cnt_out.at[pl.ds(row0, CROWS)])

            @pl.when(cnt_cond & (s == NT - 1))
            def _():
                pltpu.sync_copy(cnt_acc.at[pl.ds(row0, CROWS_L)],
                                cnt_out.at[pl.ds(row0, CROWS_L)])
        plsc.subcore_barrier()


def _aggregate_rel(h2, ei, cnt_core):
    z2d = jnp.zeros((ZR2, CW), jnp.float32)
    z1d = jnp.zeros((CZROWS,), jnp.float32)
    o1d = jnp.ones((BATCH,), jnp.float32)
    mesh = plsc.VectorSubcoreMesh(core_axis_name="c", subcore_axis_name="s")
    scratch = [
        pltpu.VMEM((WE,), jnp.int32),
        pltpu.VMEM((WE,), jnp.int32),
        pltpu.VMEM((WE,), jnp.int32),
        pltpu.VMEM((WE,), jnp.int32),
        pltpu.VMEM((KINF, BATCH, CW), jnp.float32),
    ] + [pltpu.VMEM((BATCH,), jnp.int32) for _ in range(2 * KINF)] + [
        pltpu.VMEM((REM,), jnp.int32),
        pltpu.VMEM((REM,), jnp.int32),
        pltpu.VMEM((REM, CW), jnp.float32),
        pltpu.VMEM((BATCH,), jnp.float32),
        pltpu.VMEM((ZR2, CW), jnp.float32),
        pltpu.VMEM((CZROWS,), jnp.float32),
        pltpu.VMEM_SHARED((M, CW), jnp.float32),
        pltpu.VMEM_SHARED((M,), jnp.float32),
        pltpu.SemaphoreType.DMA,
        pltpu.SemaphoreType.DMA,
    ]
    out_type = [
        jax.ShapeDtypeStruct((M2, H), jnp.float32),
        jax.ShapeDtypeStruct((M2,), jnp.float32),
    ]
    f = pl.kernel(_make_agg_body(cnt_core), mesh=mesh, out_type=out_type,
                  scratch_types=scratch,
                  compiler_params=pltpu.CompilerParams(
                      use_tc_tiling_on_sc=False))
    return f(h2, ei.reshape(2 * E), z2d, z1d, o1d)


def _cnt_col(c_ref):
    slab = c_ref[...]
    ra = lax.broadcasted_iota(jnp.int32, (BM, 16), 0) // 128
    ca = lax.broadcasted_iota(jnp.int32, (BM, 16), 1)
    ea = (ra == ca).astype(jnp.float32)
    t = jnp.dot(ea, slab, preferred_element_type=jnp.float32)
    rb = lax.broadcasted_iota(jnp.int32, (BM, H), 0) % 128
    cb = lax.broadcasted_iota(jnp.int32, (BM, H), 1)
    sel = (rb == cb).astype(jnp.float32)
    return jnp.sum(t * sel, axis=1, keepdims=True)


def _comb_body(s1_ref, c1_ref, s2_ref, c2_ref, h_ref, wr1_ref, wr2_ref,
               a_ref, wc1_ref, bc1_ref, wc2_ref, bc2_ref, o_ref):
    acc1 = jnp.dot(s1_ref[...], wr1_ref[...],
                   preferred_element_type=jnp.float32)
    acc2 = jnp.dot(s2_ref[...], wr2_ref[...],
                   preferred_element_type=jnp.float32)
    cnt1 = _cnt_col(c1_ref)
    cnt2 = _cnt_col(c2_ref)
    msg1 = jnp.where(cnt1 > 0, acc1 / jnp.maximum(cnt1, 1.0), 0.0)
    msg2 = jnp.where(cnt2 > 0, acc2 / jnp.maximum(cnt2, 1.0), 0.0)
    a1 = jnp.dot(msg1, a_ref[...], preferred_element_type=jnp.float32)
    a2 = jnp.dot(msg2, a_ref[...], preferred_element_type=jnp.float32)
    mx = jnp.maximum(a1, a2)
    e1 = jnp.exp(a1 - mx)
    e2 = jnp.exp(a2 - mx)
    inv = 1.0 / (e1 + e2)
    hcomb = msg1 * (e1 * inv) + msg2 * (e2 * inv) + h_ref[...]
    t1 = _gelu(
        jnp.dot(hcomb, wc1_ref[...], preferred_element_type=jnp.float32)
        + bc1_ref[...]
    )
    o_ref[...] = (
        jnp.dot(t1, wc2_ref[...], preferred_element_type=jnp.float32)
        + bc2_ref[...]
    )


def _combine(sum1, cnt1, sum2, cnt2, h_item, wr1, wr2, a_item, wc1, bc1,
             wc2, bc2):
    return pl.pallas_call(
        _comb_body,
        grid=(M2 // BM,),
        in_specs=[
            pl.BlockSpec((BM, H), lambda i: (i, 0)),
            pl.BlockSpec((BM // H, H), lambda i: (i, 0)),
            pl.BlockSpec((BM, H), lambda i: (i, 0)),
            pl.BlockSpec((BM // H, H), lambda i: (i, 0)),
            pl.BlockSpec((BM, H), lambda i: (i, 0)),
            pl.BlockSpec((H, H), lambda i: (0, 0)),
            pl.BlockSpec((H, H), lambda i: (0, 0)),
            pl.BlockSpec((H, 1), lambda i: (0, 0)),
            pl.BlockSpec((H, H), lambda i: (0, 0)),
            pl.BlockSpec((1, H), lambda i: (0, 0)),
            pl.BlockSpec((H, OUT), lambda i: (0, 0)),
            pl.BlockSpec((1, OUT), lambda i: (0, 0)),
        ],
        out_specs=pl.BlockSpec((BM, OUT), lambda i: (i, 0)),
        out_shape=jax.ShapeDtypeStruct((M, OUT), jnp.float32),
    )(sum1, cnt1.reshape(M2 // H, H), sum2, cnt2.reshape(M2 // H, H),
      h_item, wr1, wr2, a_item, wc1, bc1.reshape(1, H), wc2,
      bc2.reshape(1, OUT))


def kernel(x_user, x_item, ei_rates, ei_rated_by, ei_similar,
           Wu1, bu1, Wu2, bu2, Wi1, bi1, Wi2, bi2,
           Wr_rates, Wr_ratedby, Wr_similar, a_item,
           Wc1, bc1, Wc2, bc2):
    del ei_rated_by, Wr_ratedby
    h_user = _encode(x_user, Wu1, bu1, Wu2, bu2)
    sum1, cnt1 = _aggregate_rel(h_user.reshape(M2 * RPN, CW), ei_rates, 0)
    h_item = _encode(x_item, Wi1, bi1, Wi2, bi2)
    sum2, cnt2 = _aggregate_rel(h_item.reshape(M2 * RPN, CW), ei_similar, 1)
    return _combine(sum1, cnt1, sum2, cnt2, h_item,
                    Wr_rates, Wr_similar, a_item, Wc1, bc1, Wc2, bc2)
